# Initial kernel scaffold; baseline (speedup 1.0000x reference)
import jax
import jax.numpy as jnp
from jax.experimental import pallas as pl


def kernel(x, device, anchors_index):
    raise NotImplementedError("write your pallas kernel here")


# speedup vs baseline: 1.2319x; 1.2319x over previous
import jax
import jax.numpy as jnp
from jax.experimental import pallas as pl


def _body(x_ref, o_ref):
    v = x_ref[0]
    g = jax.lax.broadcasted_iota(jnp.int32, (1, 5776), 1)
    xoff = (g % 76).astype(jnp.float32)
    yoff = (g // 76).astype(jnp.float32)
    sig = jax.nn.sigmoid(v)
    ex = jnp.exp(v)
    c = jax.lax.broadcasted_iota(jnp.int32, (15, 1), 0)
    k = c % 5
    a = c // 5
    mx = (k == 0).astype(jnp.float32)
    my = (k == 1).astype(jnp.float32)
    mc = (k == 4).astype(jnp.float32)
    mwh = jnp.logical_or(k == 2, k == 3).astype(jnp.float32)
    aw = jnp.where(a == 0, 10.0, jnp.where(a == 1, 16.0, 33.0))
    ah = jnp.where(a == 0, 13.0, jnp.where(a == 1, 30.0, 23.0))
    anc = jnp.where(k == 2, aw, ah).astype(jnp.float32)
    res = (mx * (sig + xoff) * 8.0 + my * (sig + yoff) * 8.0
           + mc * sig + mwh * ex * anc)
    resp = jnp.concatenate([res, jnp.zeros((1, 5776), jnp.float32)], axis=0)
    o_ref[0] = resp.T[:, :15]


def kernel(x, device, anchors_index):
    b = x.shape[0]
    xf = x.reshape(b, 15, 5776)
    y = pl.pallas_call(
        _body,
        grid=(b,),
        in_specs=[pl.BlockSpec((1, 15, 5776), lambda i: (i, 0, 0))],
        out_specs=pl.BlockSpec((1, 5776, 15), lambda i: (i, 0, 0)),
        out_shape=jax.ShapeDtypeStruct((b, 5776, 15), jnp.float32),
    )(xf)
    xx = y.reshape(b, 17328, 5)
    heat = y.reshape(b, 76, 76, 3, 5)
    return heat, xx
